# initial kernel scaffold (unmeasured)
import jax
import jax.numpy as jnp
from jax import lax
from jax.experimental import pallas as pl
from jax.experimental.pallas import tpu as pltpu


def kernel(Q, K, V):
    b, sq, h, d = Q.shape
    skv = K.shape[1]
    scale = d ** -0.5

    def partial_body(q_ref, k_ref, v_ref, u_ref, l_ref):
        q = q_ref[0, 0]
        k = k_ref[0]
        v = v_ref[0]
        s = lax.dot_general(
            q, k,
            dimension_numbers=(((1,), (2,)), ((0,), (1,))),
            preferred_element_type=jnp.float32,
        ) * scale
        p = jnp.exp(s)
        l_ref[0] = jnp.sum(p, axis=1)
        u_ref[0, 0] = lax.dot_general(
            p, v,
            dimension_numbers=(((1,), (0,)), ((0,), (1,))),
            preferred_element_type=jnp.float32,
        )

    U, L = pl.pallas_call(
        partial_body,
        grid=(b,),
        in_specs=[
            pl.BlockSpec((1, sq, h, d), lambda i: (i, 0, 0, 0)),
            pl.BlockSpec((1, skv, h, d), lambda i: (i, 0, 0, 0)),
            pl.BlockSpec((1, skv, h, d), lambda i: (i, 0, 0, 0)),
        ],
        out_specs=[
            pl.BlockSpec((1, sq, h, d), lambda i: (i, 0, 0, 0)),
            pl.BlockSpec((1, h), lambda i: (i, 0)),
        ],
        out_shape=[
            jax.ShapeDtypeStruct((b, sq, h, d), jnp.float32),
            jax.ShapeDtypeStruct((b, h), jnp.float32),
        ],
    )(Q, K, V)

    def merge_body(u_ref, l_ref, o_ref, u_peer, l_peer, send_sems, recv_sems):
        my_x = lax.axis_index("x")
        my_y = lax.axis_index("y")
        my_z = lax.axis_index("z")
        partner = (1 - my_x, my_y, my_z)

        barrier = pltpu.get_barrier_semaphore()
        pl.semaphore_signal(
            barrier, inc=1,
            device_id=partner, device_id_type=pl.DeviceIdType.MESH,
        )
        pl.semaphore_wait(barrier, 1)

        ru = pltpu.make_async_remote_copy(
            src_ref=u_ref, dst_ref=u_peer,
            send_sem=send_sems.at[0], recv_sem=recv_sems.at[0],
            device_id=partner, device_id_type=pl.DeviceIdType.MESH,
        )
        rl = pltpu.make_async_remote_copy(
            src_ref=l_ref, dst_ref=l_peer,
            send_sem=send_sems.at[1], recv_sem=recv_sems.at[1],
            device_id=partner, device_id_type=pl.DeviceIdType.MESH,
        )
        ru.start()
        rl.start()
        ru.wait()
        rl.wait()

        u_tot = u_ref[...] + u_peer[...]
        l_tot = l_ref[...] + l_peer[...]
        o_ref[...] = u_tot / l_tot.reshape(b, 1, h, 1)

    return pl.pallas_call(
        merge_body,
        in_specs=[
            pl.BlockSpec(memory_space=pltpu.VMEM),
            pl.BlockSpec(memory_space=pltpu.VMEM),
        ],
        out_specs=pl.BlockSpec(memory_space=pltpu.VMEM),
        out_shape=jax.ShapeDtypeStruct((b, sq, h, d), jnp.float32),
        scratch_shapes=[
            pltpu.VMEM((b, sq, h, d), jnp.float32),
            pltpu.VMEM((b, h), jnp.float32),
            pltpu.SemaphoreType.DMA((2,)),
            pltpu.SemaphoreType.DMA((2,)),
        ],
        compiler_params=pltpu.CompilerParams(collective_id=0),
    )(U, L)


# baseline (device time: 253664 ns/iter reference)
import jax
import jax.numpy as jnp
from jax import lax
from jax.experimental import pallas as pl
from jax.experimental.pallas import tpu as pltpu


def kernel(Q, K, V):
    b, sq, h, d = Q.shape
    skv = K.shape[1]
    scale = d ** -0.5

    hd = h * d
    hpb = 128 // d
    Qr = Q.reshape(b, sq, hd)
    Kr = K.reshape(b, skv, hd)
    Vr = V.reshape(b, skv, hd)

    def partial_body(q_ref, k_ref, v_ref, u_ref, l_ref):
        q2 = q_ref[0]
        k2 = k_ref[0]
        v2 = v_ref[0]
        us = []
        ls = []
        for hh in range(hpb):
            sl = slice(hh * d, (hh + 1) * d)
            s = lax.dot_general(
                q2[:, sl], k2[:, sl],
                dimension_numbers=(((1,), (1,)), ((), ())),
                preferred_element_type=jnp.float32,
            ) * scale
            p = jnp.exp(s)
            ls.append(jnp.broadcast_to(
                jnp.sum(p, axis=1, keepdims=True), (1, d)
            ))
            us.append(lax.dot_general(
                p, v2[:, sl],
                dimension_numbers=(((1,), (0,)), ((), ())),
                preferred_element_type=jnp.float32,
            ))
        u_ref[0] = jnp.concatenate(us, axis=1)
        l_ref[0] = jnp.concatenate(ls, axis=1)

    U, L = pl.pallas_call(
        partial_body,
        grid=(b, h // hpb),
        in_specs=[
            pl.BlockSpec((1, sq, hpb * d), lambda i, j: (i, 0, j)),
            pl.BlockSpec((1, skv, hpb * d), lambda i, j: (i, 0, j)),
            pl.BlockSpec((1, skv, hpb * d), lambda i, j: (i, 0, j)),
        ],
        out_specs=[
            pl.BlockSpec((1, sq, hpb * d), lambda i, j: (i, 0, j)),
            pl.BlockSpec((1, sq, hpb * d), lambda i, j: (i, 0, j)),
        ],
        out_shape=[
            jax.ShapeDtypeStruct((b, sq, hd), jnp.float32),
            jax.ShapeDtypeStruct((b, sq, hd), jnp.float32),
        ],
    )(Qr, Kr, Vr)

    def merge_body(u_ref, l_ref, o_ref, u_peer, l_peer, send_sems, recv_sems):
        my_x = lax.axis_index("x")
        my_y = lax.axis_index("y")
        my_z = lax.axis_index("z")
        partner = (1 - my_x, my_y, my_z)

        barrier = pltpu.get_barrier_semaphore()
        pl.semaphore_signal(
            barrier, inc=1,
            device_id=partner, device_id_type=pl.DeviceIdType.MESH,
        )
        pl.semaphore_wait(barrier, 1)

        ru = pltpu.make_async_remote_copy(
            src_ref=u_ref, dst_ref=u_peer,
            send_sem=send_sems.at[0], recv_sem=recv_sems.at[0],
            device_id=partner, device_id_type=pl.DeviceIdType.MESH,
        )
        rl = pltpu.make_async_remote_copy(
            src_ref=l_ref, dst_ref=l_peer,
            send_sem=send_sems.at[1], recv_sem=recv_sems.at[1],
            device_id=partner, device_id_type=pl.DeviceIdType.MESH,
        )
        ru.start()
        rl.start()
        ru.wait()
        rl.wait()

        u_tot = u_ref[...] + u_peer[...]
        l_tot = l_ref[...] + l_peer[...]
        o_ref[...] = u_tot / l_tot

    O = pl.pallas_call(
        merge_body,
        in_specs=[
            pl.BlockSpec(memory_space=pltpu.VMEM),
            pl.BlockSpec(memory_space=pltpu.VMEM),
        ],
        out_specs=pl.BlockSpec(memory_space=pltpu.VMEM),
        out_shape=jax.ShapeDtypeStruct((b, sq, hd), jnp.float32),
        scratch_shapes=[
            pltpu.VMEM((b, sq, hd), jnp.float32),
            pltpu.VMEM((b, sq, hd), jnp.float32),
            pltpu.SemaphoreType.DMA((2,)),
            pltpu.SemaphoreType.DMA((2,)),
        ],
        compiler_params=pltpu.CompilerParams(collective_id=0),
    )(U, L)
    return O.reshape(b, sq, h, d)


# device time: 193671 ns/iter; 1.3098x vs baseline; 1.3098x over previous
import jax
import jax.numpy as jnp
from jax import lax
from jax.experimental import pallas as pl
from jax.experimental.pallas import tpu as pltpu


def kernel(Q, K, V):
    b, sq, h, d = Q.shape
    skv = K.shape[1]
    scale = d ** -0.5

    hd = h * d
    hpb = 8
    width = hpb * d
    Qr = Q.reshape(b, sq, hd)
    Kr = K.reshape(b, skv, hd)
    Vr = V.reshape(b, skv, hd)

    def partial_body(q_ref, k_ref, v_ref, u_ref, l_ref):
        q2 = q_ref[0]
        k2 = k_ref[0]
        v2 = v_ref[0]
        ci = lax.broadcasted_iota(jnp.int32, (hpb, width), 1)
        hi = lax.broadcasted_iota(jnp.int32, (hpb, width), 0)
        mask = (ci // d == hi).astype(jnp.float32)
        qt = q2 * mask
        s = lax.dot_general(
            k2, qt,
            dimension_numbers=(((1,), (1,)), ((), ())),
            preferred_element_type=jnp.float32,
        ) * scale
        p = jnp.exp(s)
        pexp = lax.dot_general(
            p, mask,
            dimension_numbers=(((1,), (0,)), ((), ())),
            preferred_element_type=jnp.float32,
        )
        u_ref[0] = jnp.sum(pexp * v2, axis=0, keepdims=True)
        l_ref[0] = jnp.sum(pexp, axis=0, keepdims=True)

    U, L = pl.pallas_call(
        partial_body,
        grid=(b, h // hpb),
        in_specs=[
            pl.BlockSpec((1, sq, hpb * d), lambda i, j: (i, 0, j)),
            pl.BlockSpec((1, skv, hpb * d), lambda i, j: (i, 0, j)),
            pl.BlockSpec((1, skv, hpb * d), lambda i, j: (i, 0, j)),
        ],
        out_specs=[
            pl.BlockSpec((1, sq, hpb * d), lambda i, j: (i, 0, j)),
            pl.BlockSpec((1, sq, hpb * d), lambda i, j: (i, 0, j)),
        ],
        out_shape=[
            jax.ShapeDtypeStruct((b, sq, hd), jnp.float32),
            jax.ShapeDtypeStruct((b, sq, hd), jnp.float32),
        ],
    )(Qr, Kr, Vr)

    def merge_body(u_ref, l_ref, o_ref, u_peer, l_peer, send_sems, recv_sems):
        my_x = lax.axis_index("x")
        my_y = lax.axis_index("y")
        my_z = lax.axis_index("z")
        partner = (1 - my_x, my_y, my_z)

        barrier = pltpu.get_barrier_semaphore()
        pl.semaphore_signal(
            barrier, inc=1,
            device_id=partner, device_id_type=pl.DeviceIdType.MESH,
        )
        pl.semaphore_wait(barrier, 1)

        ru = pltpu.make_async_remote_copy(
            src_ref=u_ref, dst_ref=u_peer,
            send_sem=send_sems.at[0], recv_sem=recv_sems.at[0],
            device_id=partner, device_id_type=pl.DeviceIdType.MESH,
        )
        rl = pltpu.make_async_remote_copy(
            src_ref=l_ref, dst_ref=l_peer,
            send_sem=send_sems.at[1], recv_sem=recv_sems.at[1],
            device_id=partner, device_id_type=pl.DeviceIdType.MESH,
        )
        ru.start()
        rl.start()
        ru.wait()
        rl.wait()

        u_tot = u_ref[...] + u_peer[...]
        l_tot = l_ref[...] + l_peer[...]
        o_ref[...] = u_tot / l_tot

    O = pl.pallas_call(
        merge_body,
        in_specs=[
            pl.BlockSpec(memory_space=pltpu.VMEM),
            pl.BlockSpec(memory_space=pltpu.VMEM),
        ],
        out_specs=pl.BlockSpec(memory_space=pltpu.VMEM),
        out_shape=jax.ShapeDtypeStruct((b, sq, hd), jnp.float32),
        scratch_shapes=[
            pltpu.VMEM((b, sq, hd), jnp.float32),
            pltpu.VMEM((b, sq, hd), jnp.float32),
            pltpu.SemaphoreType.DMA((2,)),
            pltpu.SemaphoreType.DMA((2,)),
        ],
        compiler_params=pltpu.CompilerParams(collective_id=0),
    )(U, L)
    return O.reshape(b, sq, h, d)
